# baseline (speedup 1.0000x reference)
import jax
import jax.numpy as jnp
from jax.experimental import pallas as pl
from jax.experimental.pallas import tpu as pltpu


def _make_kernel(B, N, d_in, d_hid, d_out, CH):
    NCH = N // CH
    TOT = B * NCH

    def _gcn_kernel(x_hbm, adj_hbm, w1_ref, b1_ref, w2_ref, b2_ref,
                    out_ref, abuf, xbuf, tbuf, gbuf, adma, xdma):
        def adj_copy(i):
            return pltpu.make_async_copy(
                adj_hbm.at[pl.ds(i * CH, CH), :], abuf.at[i % 3],
                adma.at[i % 3])

        def x_copy(b):
            return pltpu.make_async_copy(
                x_hbm.at[pl.ds(b * N, N), :], xbuf.at[b % 2],
                xdma.at[b % 2])

        w1 = w1_ref[...]
        b1v = b1_ref[...]

        x_copy(0).start()
        adj_copy(0).start()
        adj_copy(1).start()
        adj_copy(2).start()
        x_copy(0).wait()
        tbuf[0] = jnp.dot(xbuf[0], w1, preferred_element_type=jnp.float32)

        for b in range(B):
            t = tbuf[b % 2]
            cs = None
            for c in range(NCH):
                i = b * NCH + c
                adj_copy(i).wait()
                chunk = abuf[i % 3]
                h = jnp.dot(chunk, t,
                            preferred_element_type=jnp.float32) + b1v
                gbuf[pl.ds(c * CH, CH), :] = jnp.maximum(h, 0.01 * h)
                s = jnp.sum(chunk, axis=0)
                cs = s if cs is None else cs + s
                if i + 3 < TOT:
                    adj_copy(i + 3).start()
                if c == 0 and b + 1 < B:
                    x_copy(b + 1).start()
                if c == NCH - 2 and b + 1 < B:
                    x_copy(b + 1).wait()
                    tbuf[(b + 1) % 2] = jnp.dot(
                        xbuf[(b + 1) % 2], w1,
                        preferred_element_type=jnp.float32)
            r = cs * (1.0 / N)
            v = jnp.sum(gbuf[...] * r[:, None], axis=0)
            out_ref[b] = (jnp.dot(v[None, :], w2_ref[...],
                                  preferred_element_type=jnp.float32)
                          + b2_ref[...])

    return _gcn_kernel


def kernel(x, graph_batch, W1, b1, W2, b2):
    B, N, d_in = x.shape
    d_hid = W1.shape[1]
    d_out = W2.shape[1]
    CH = 256
    x2 = x.reshape(B * N, d_in)
    adj2 = graph_batch.reshape(B * N, N)
    b1r = b1.reshape(1, d_hid)
    b2r = b2.reshape(1, d_out)
    return pl.pallas_call(
        _make_kernel(B, N, d_in, d_hid, d_out, CH),
        grid=(1,),
        in_specs=[
            pl.BlockSpec(memory_space=pl.ANY),
            pl.BlockSpec(memory_space=pl.ANY),
            pl.BlockSpec((d_in, d_hid), lambda i: (0, 0)),
            pl.BlockSpec((1, d_hid), lambda i: (0, 0)),
            pl.BlockSpec((d_hid, d_out), lambda i: (0, 0)),
            pl.BlockSpec((1, d_out), lambda i: (0, 0)),
        ],
        out_specs=pl.BlockSpec((B, 1, d_out), lambda i: (0, 0, 0)),
        out_shape=jax.ShapeDtypeStruct((B, 1, d_out), jnp.float32),
        scratch_shapes=[
            pltpu.VMEM((3, CH, N), jnp.float32),
            pltpu.VMEM((2, N, d_in), jnp.float32),
            pltpu.VMEM((2, N, d_hid), jnp.float32),
            pltpu.VMEM((N, d_hid), jnp.float32),
            pltpu.SemaphoreType.DMA((3,)),
            pltpu.SemaphoreType.DMA((2,)),
        ],
    )(x2, adj2, W1, b1r, W2, b2r).reshape(B, d_out)


# speedup vs baseline: 1.3309x; 1.3309x over previous
import jax
import jax.numpy as jnp
from jax.experimental import pallas as pl


def _gcn_kernel(x_ref, adj_ref, w1_ref, b1_ref, w2_ref, b2_ref, out_ref):
    adj = adj_ref[0]
    t = jnp.dot(x_ref[0].astype(jnp.bfloat16),
                w1_ref[...].astype(jnp.bfloat16),
                preferred_element_type=jnp.float32)
    h = jnp.dot(adj.astype(jnp.bfloat16), t.astype(jnp.bfloat16),
                preferred_element_type=jnp.float32) + b1_ref[...]
    g = jnp.maximum(h, 0.01 * h)
    n = adj.shape[0]
    r = jnp.sum(adj, axis=0) * (1.0 / n)
    v = jnp.sum(g * r[:, None], axis=0)
    out_ref[0] = (jnp.dot(v[None, :], w2_ref[...],
                          preferred_element_type=jnp.float32)
                  + b2_ref[...])


def kernel(x, graph_batch, W1, b1, W2, b2):
    B, N, d_in = x.shape
    d_hid = W1.shape[1]
    d_out = W2.shape[1]
    b1r = b1.reshape(1, d_hid)
    b2r = b2.reshape(1, d_out)
    return pl.pallas_call(
        _gcn_kernel,
        grid=(B,),
        in_specs=[
            pl.BlockSpec((1, N, d_in), lambda b: (b, 0, 0)),
            pl.BlockSpec((1, N, N), lambda b: (b, 0, 0)),
            pl.BlockSpec((d_in, d_hid), lambda b: (0, 0)),
            pl.BlockSpec((1, d_hid), lambda b: (0, 0)),
            pl.BlockSpec((d_hid, d_out), lambda b: (0, 0)),
            pl.BlockSpec((1, d_out), lambda b: (0, 0)),
        ],
        out_specs=pl.BlockSpec((1, 1, d_out), lambda b: (b, 0, 0)),
        out_shape=jax.ShapeDtypeStruct((B, 1, d_out), jnp.float32),
    )(x, graph_batch, W1, b1r, W2, b2r).reshape(B, d_out)


# speedup vs baseline: 1.3439x; 1.0097x over previous
import jax
import jax.numpy as jnp
from jax.experimental import pallas as pl


def _gcn_kernel(x_ref, adj_ref, w1_ref, b1_ref, w2_ref, b2_ref, out_ref):
    adj = adj_ref[0]
    t = jnp.dot(x_ref[0], w1_ref[...],
                preferred_element_type=jnp.float32)
    h = jnp.dot(adj, t, preferred_element_type=jnp.float32) + b1_ref[...]
    g = jnp.where(h >= 0.0, h, 0.01 * h)
    n = adj.shape[0]
    r = jnp.sum(adj, axis=0) * (1.0 / n)
    v = jnp.sum(g * r[:, None], axis=0)
    out_ref[0] = (jnp.dot(v[None, :], w2_ref[...],
                          preferred_element_type=jnp.float32)
                  + b2_ref[...])


def kernel(x, graph_batch, W1, b1, W2, b2):
    B, N, d_in = x.shape
    d_hid = W1.shape[1]
    d_out = W2.shape[1]
    b1r = b1.reshape(1, d_hid)
    b2r = b2.reshape(1, d_out)
    return pl.pallas_call(
        _gcn_kernel,
        grid=(B,),
        in_specs=[
            pl.BlockSpec((1, N, d_in), lambda b: (b, 0, 0)),
            pl.BlockSpec((1, N, N), lambda b: (b, 0, 0)),
            pl.BlockSpec((d_in, d_hid), lambda b: (0, 0)),
            pl.BlockSpec((1, d_hid), lambda b: (0, 0)),
            pl.BlockSpec((d_hid, d_out), lambda b: (0, 0)),
            pl.BlockSpec((1, d_out), lambda b: (0, 0)),
        ],
        out_specs=pl.BlockSpec((1, 1, d_out), lambda b: (b, 0, 0)),
        out_shape=jax.ShapeDtypeStruct((B, 1, d_out), jnp.float32),
    )(x, graph_batch, W1, b1r, W2, b2r).reshape(B, d_out)
